# baseline (speedup 1.0000x reference)
import functools

import jax
import jax.numpy as jnp
from jax import lax
from jax.experimental import pallas as pl
from jax.experimental.pallas import tpu as pltpu
from jax.experimental.pallas import tpu_sc as plsc

N = 10000
E = 320000
D = 128
H = 128
C = 10
G = 128
EPS = 1e-5

NC = 2
NS = 16
NW = NC * NS
NPAD = 10240
EW = E // NW
K = 80
STEPS = EW // K
MSTEPS = 128
PH = 64
EPAD = NW * MSTEPS * K


def _scale_rows(rows, sall, i):
    @pl.loop(0, K // 16)
    def _(t):
        s16 = sall[i, pl.ds(t * 16, 16)]
        for l in range(16):
            sc = s16[l]
            for j in range(8):
                rows[t * 16 + l, pl.ds(j * 16, 16)] = (
                    rows[t * 16 + l, pl.ds(j * 16, 16)] * sc)


def _msg_body(scaled, *refs):
    if scaled:
        (h_hbm, row3, col3, ew3, out_hbm, acc_sh,
         ridx_all, cidx_all, sall, rows_a, rows_b,
         gsem_a, gsem_b, ssem_a, ssem_b) = refs
    else:
        (h_hbm, row3, col3, out_hbm, acc_sh,
         ridx_all, cidx_all, rows_a, rows_b,
         gsem_a, gsem_b, ssem_a, ssem_b) = refs
    cid = lax.axis_index("c")
    sid = lax.axis_index("s")
    wid = sid * NC + cid
    @pl.loop(0, K)
    def _(r):
        for j in range(8):
            rows_a[r, pl.ds(j * 16, 16)] = jnp.zeros((16,), jnp.float32)
    slab = NPAD // NS
    @pl.loop(0, slab // K)
    def _(t):
        pltpu.sync_copy(rows_a, acc_sh.at[pl.ds(sid * slab + t * K, K), :])
    plsc.subcore_barrier()

    def phase(s0, nsteps):
        pltpu.sync_copy(row3.at[wid, pl.ds(s0, nsteps)],
                        ridx_all.at[pl.ds(0, nsteps)])
        pltpu.sync_copy(col3.at[wid, pl.ds(s0, nsteps)],
                        cidx_all.at[pl.ds(0, nsteps)])
        if scaled:
            pltpu.sync_copy(ew3.at[wid, pl.ds(s0, nsteps)],
                            sall.at[pl.ds(0, nsteps)])
        @pl.loop(0, nsteps // 2)
        def _(ip):
            i0 = ip * 2
            i1 = i0 + 1
            pltpu.async_copy(h_hbm.at[ridx_all.at[i0]], rows_a, gsem_a).wait()
            if scaled:
                _scale_rows(rows_a, sall, i0)
            pltpu.sync_copy(rows_a, acc_sh.at[cidx_all.at[i0]], add=True)
            pltpu.async_copy(h_hbm.at[ridx_all.at[i1]], rows_b, gsem_b).wait()
            if scaled:
                _scale_rows(rows_b, sall, i1)
            pltpu.sync_copy(rows_b, acc_sh.at[cidx_all.at[i1]], add=True)
        if nsteps % 2:
            it = nsteps - 1
            pltpu.async_copy(h_hbm.at[ridx_all.at[it]], rows_a, gsem_a).wait()
            if scaled:
                _scale_rows(rows_a, sall, it)
            pltpu.sync_copy(rows_a, acc_sh.at[cidx_all.at[it]], add=True)

    phase(0, PH)
    phase(PH, MSTEPS - PH)
    plsc.subcore_barrier()
    pltpu.sync_copy(acc_sh.at[pl.ds(sid * slab, slab), :],
                    out_hbm.at[cid, pl.ds(sid * slab, slab), :])


def _attn_body(u_hbm, v_hbm, row_hbm, col_hbm,
               ewc_hbm, ewo_hbm, deg_hbm,
               u_vmem, v_vmem, dc_vmem, do_vmem, ridx, cidx, wc_buf, wo_buf):
    cid = lax.axis_index("c")
    sid = lax.axis_index("s")
    wid = sid * NC + cid
    pltpu.sync_copy(u_hbm, u_vmem)
    pltpu.sync_copy(v_hbm, v_vmem)
    @pl.loop(0, NPAD // 16)
    def _(t):
        dc_vmem[pl.ds(t * 16, 16)] = jnp.zeros((16,), jnp.float32)
        do_vmem[pl.ds(t * 16, 16)] = jnp.zeros((16,), jnp.float32)
    base0 = wid * EW
    @pl.loop(0, STEPS)
    def _(i):
        base = base0 + i * K
        pltpu.sync_copy(row_hbm.at[pl.ds(base, K)], ridx)
        pltpu.sync_copy(col_hbm.at[pl.ds(base, K)], cidx)
        @pl.loop(0, K // 16)
        def _(t):
            r16 = ridx[pl.ds(t * 16, 16)]
            c16 = cidx[pl.ds(t * 16, 16)]
            s = plsc.load_gather(u_vmem, [r16]) + plsc.load_gather(v_vmem, [c16])
            wc = 1.0 / (1.0 + jnp.exp(-s))
            wo = 1.0 - wc
            wc_buf[pl.ds(t * 16, 16)] = wc
            wo_buf[pl.ds(t * 16, 16)] = wo
            plsc.addupdate_scatter(dc_vmem, [r16], wc)
            plsc.addupdate_scatter(do_vmem, [r16], wo)
        pltpu.sync_copy(wc_buf, ewc_hbm.at[pl.ds(base, K)])
        pltpu.sync_copy(wo_buf, ewo_hbm.at[pl.ds(base, K)])
    pltpu.sync_copy(dc_vmem, deg_hbm.at[wid, 0])
    pltpu.sync_copy(do_vmem, deg_hbm.at[wid, 1])


@jax.jit
def _sc_attn(u_pad, v_pad, row, col):
    mesh = plsc.VectorSubcoreMesh(core_axis_name="c", subcore_axis_name="s")
    kern = pl.kernel(
        _attn_body,
        compiler_params=pltpu.CompilerParams(needs_layout_passes=False),
        out_type=(
            jax.ShapeDtypeStruct((E,), jnp.float32),
            jax.ShapeDtypeStruct((E,), jnp.float32),
            jax.ShapeDtypeStruct((NW, 2, NPAD), jnp.float32),
        ),
        mesh=mesh,
        scratch_types=[
            pltpu.VMEM((NPAD,), jnp.float32),
            pltpu.VMEM((NPAD,), jnp.float32),
            pltpu.VMEM((NPAD,), jnp.float32),
            pltpu.VMEM((NPAD,), jnp.float32),
            pltpu.VMEM((K,), jnp.int32),
            pltpu.VMEM((K,), jnp.int32),
            pltpu.VMEM((K,), jnp.float32),
            pltpu.VMEM((K,), jnp.float32),
        ],
    )
    return kern(u_pad, v_pad, row, col)


@jax.jit
def _sc_msgpass(h_pad, row, col):
    mesh = plsc.VectorSubcoreMesh(core_axis_name="c", subcore_axis_name="s")
    kern = pl.kernel(
        functools.partial(_msg_body, False),
        out_type=jax.ShapeDtypeStruct((NC, NPAD, 128), jnp.float32),
        mesh=mesh,
        scratch_types=[
            pltpu.VMEM_SHARED((NPAD, 128), jnp.float32),
            pltpu.VMEM((PH, K), jnp.int32),
            pltpu.VMEM((PH, K), jnp.int32),
            pltpu.VMEM((K, 128), jnp.float32),
            pltpu.VMEM((K, 128), jnp.float32),
            pltpu.SemaphoreType.DMA,
            pltpu.SemaphoreType.DMA,
            pltpu.SemaphoreType.DMA,
            pltpu.SemaphoreType.DMA,
        ],
    )
    return kern(h_pad, row.reshape(NW, MSTEPS, K), col.reshape(NW, MSTEPS, K))


@jax.jit
def _sc_msgpass_scaled(h_pad, row, col, ew):
    mesh = plsc.VectorSubcoreMesh(core_axis_name="c", subcore_axis_name="s")
    kern = pl.kernel(
        functools.partial(_msg_body, True),
        out_type=jax.ShapeDtypeStruct((NC, NPAD, 128), jnp.float32),
        mesh=mesh,
        scratch_types=[
            pltpu.VMEM_SHARED((NPAD, 128), jnp.float32),
            pltpu.VMEM((PH, K), jnp.int32),
            pltpu.VMEM((PH, K), jnp.int32),
            pltpu.VMEM((PH, K), jnp.float32),
            pltpu.VMEM((K, 128), jnp.float32),
            pltpu.VMEM((K, 128), jnp.float32),
            pltpu.SemaphoreType.DMA,
            pltpu.SemaphoreType.DMA,
            pltpu.SemaphoreType.DMA,
            pltpu.SemaphoreType.DMA,
        ],
    )
    return kern(h_pad, row.reshape(NW, MSTEPS, K), col.reshape(NW, MSTEPS, K),
                ew.reshape(NW, MSTEPS, K))


def _bn(x, g, b):
    m = jnp.mean(x, axis=0)
    v = jnp.mean(x * x, axis=0) - m * m
    return (x - m) * lax.rsqrt(v + EPS) * g + b


def _log_softmax(z):
    zm = z - jnp.max(z, axis=-1, keepdims=True)
    return zm - jnp.log(jnp.sum(jnp.exp(zm), axis=-1, keepdims=True))


def _head(z, p, pre):
    z = _bn(z, p[pre + "1bn_g"], p[pre + "1bn_b"])
    z = jax.nn.relu(z @ p[pre + "1_W"] + p[pre + "1_b"])
    z = _bn(z, p[pre + "2bn_g"], p[pre + "2bn_b"])
    z = z @ p[pre + "2_W"] + p[pre + "2_b"]
    return _log_softmax(z)


def _pool_heads_body(xc_ref, xo_ref, batch_ref, *rest):
    (hp_refs, outc_ref, outo_ref, outco_ref) = (rest[:-3], rest[-3], rest[-2], rest[-1])
    names = _HEAD_PARAM_NAMES
    p = {k: r[...] for k, r in zip(names, hp_refs)}
    onehot = (batch_ref[0:1, :] == lax.broadcasted_iota(jnp.int32, (G, N), 0))
    onehot = onehot.astype(jnp.float32)
    pc = jnp.dot(onehot, xc_ref[...], preferred_element_type=jnp.float32)
    po = jnp.dot(onehot, xo_ref[...], preferred_element_type=jnp.float32)
    outc_ref[...] = _head(pc, p, "c")
    outo_ref[...] = _head(po, p, "o")
    outco_ref[...] = _head(pc + po, p, "co")


_HEAD_PARAM_NAMES = tuple(
    pre + suf
    for pre in ("c", "o", "co")
    for suf in ("1bn_g", "1bn_b", "1_W", "1_b", "2bn_g", "2bn_b", "2_W", "2_b")
)


def _pool_and_heads(xc, xo, batch, params):
    hp = [params[k] for k in _HEAD_PARAM_NAMES]
    out_shape = [jax.ShapeDtypeStruct((G, C), jnp.float32)] * 3
    outs = pl.pallas_call(
        _pool_heads_body,
        out_shape=out_shape,
    )(xc, xo, batch.reshape(1, N), *hp)
    return outs


def _pad_nodes(h):
    return jnp.pad(h, ((0, NPAD - N), (0, 0)))


def _gcn(x, row_p, col_p, ew, dis, W, b):
    h2 = x @ W
    gs = dis[:, None] * h2
    ew_p = jnp.concatenate([ew, jnp.zeros((EPAD - E,), jnp.float32)])
    mp = _sc_msgpass_scaled(_pad_nodes(gs), row_p, col_p, ew_p)
    out = dis[:, None] * (mp[0, :N] + mp[1, :N] + gs)
    return out + b


def _gin(h, row_p, col_p, p):
    mp = _sc_msgpass(_pad_nodes(h), row_p, col_p)
    h = h + mp[0, :N] + mp[1, :N]
    h = jax.nn.relu(_bn(h @ p["W1"] + p["b1"], p["g1"], p["be1"]))
    return jax.nn.relu(h @ p["W2"] + p["b2"])


def kernel(x, edge_index, batch, params):
    p = params
    row, col = edge_index[0], edge_index[1]
    pad_idx = jnp.full((EPAD - E,), NPAD - 1, jnp.int32)
    row_p = jnp.concatenate([row, pad_idx])
    col_p = jnp.concatenate([col, pad_idx])
    h = _bn(x, p["bn_feat_g"], p["bn_feat_b"])
    h = jax.nn.relu(h @ p["conv_feat_W"] + p["conv_feat_b"])
    for lp in p["gin"]:
        h = _gin(h, row_p, col_p, lp)
    wea = p["ea_W"]
    u = h @ (wea[:H, 0] - wea[:H, 1]) + (p["ea_b"][0] - p["ea_b"][1])
    v = h @ (wea[H:, 0] - wea[H:, 1])
    ewc, ewo, degtab = _sc_attn(jnp.pad(u, (0, NPAD - N)),
                                jnp.pad(v, (0, NPAD - N)), row, col)
    degsum = degtab.sum(axis=0)
    dis_c = lax.rsqrt(degsum[0, :N] + 1.0)
    dis_o = lax.rsqrt(degsum[1, :N] + 1.0)
    nl = h @ p["na_W"] + p["na_b"]
    na0 = jax.nn.sigmoid(nl[:, 0] - nl[:, 1])
    xc = na0[:, None] * h
    xo = (1.0 - na0)[:, None] * h
    xc = jax.nn.relu(_gcn(_bn(xc, p["bnc_g"], p["bnc_b"]), row_p, col_p, ewc, dis_c, p["cc_W"], p["cc_b"]))
    xo = jax.nn.relu(_gcn(_bn(xo, p["bno_g"], p["bno_b"]), row_p, col_p, ewo, dis_o, p["oc_W"], p["oc_b"]))
    outc, outo, outco = _pool_and_heads(xc, xo, batch, p)
    return (outc, outo, outco)


# speedup vs baseline: 2.4543x; 2.4543x over previous
import functools

import jax
import jax.numpy as jnp
from jax import lax
from jax.experimental import pallas as pl
from jax.experimental.pallas import tpu as pltpu
from jax.experimental.pallas import tpu_sc as plsc

N = 10000
E = 320000
D = 128
H = 128
C = 10
G = 128
EPS = 1e-5

NC = 2
NS = 16
NW = NC * NS
NPAD = 10240
EW = E // NW
K = 80
STEPS = EW // K
MSTEPS = 128
PH = 64
EPAD = NW * MSTEPS * K


def _scale_rows(rows, sbuf):
    @pl.loop(0, K // 16)
    def _(t):
        s16 = sbuf[pl.ds(t * 16, 16)]
        for l in range(16):
            sc = s16[l]
            for j in range(8):
                rows[t * 16 + l, pl.ds(j * 16, 16)] = (
                    rows[t * 16 + l, pl.ds(j * 16, 16)] * sc)


def _msg_body(scaled, *refs):
    if scaled:
        (h_hbm, row_hbm, col_hbm, ew_hbm, out_hbm, acc_sh,
         ridx_a, cidx_a, sbuf_a, ridx_b, cidx_b, sbuf_b, rows_a, rows_b,
         isem_a, isem_b, gsem_a, gsem_b, ssem_a, ssem_b) = refs
    else:
        (h_hbm, row_hbm, col_hbm, out_hbm, acc_sh,
         ridx_a, cidx_a, ridx_b, cidx_b, rows_a, rows_b,
         isem_a, isem_b, gsem_a, gsem_b, ssem_a, ssem_b) = refs
    cid = lax.axis_index("c")
    sid = lax.axis_index("s")
    wid = sid * NC + cid
    @pl.loop(0, K)
    def _(r):
        for j in range(8):
            rows_a[r, pl.ds(j * 16, 16)] = jnp.zeros((16,), jnp.float32)
    slab = NPAD // NS
    @pl.loop(0, slab // K)
    def _(t):
        pltpu.sync_copy(rows_a, acc_sh.at[pl.ds(sid * slab + t * K, K), :])
    plsc.subcore_barrier()

    def fetch_idx(base, ridx, cidx, sbuf, sem):
        ds = [pltpu.async_copy(row_hbm.at[pl.ds(base, K)], ridx, sem),
              pltpu.async_copy(col_hbm.at[pl.ds(base, K)], cidx, sem)]
        if scaled:
            ds.append(pltpu.async_copy(ew_hbm.at[pl.ds(base, K)], sbuf, sem))
        return ds

    def drain(ds):
        for d in ds:
            d.wait()

    base0 = wid * EW
    @pl.loop(0, STEPS // 2)
    def _(ip):
        base_a = base0 + ip * (2 * K)
        base_b = base_a + K
        ia = fetch_idx(base_a, ridx_a, cidx_a, sbuf_a if scaled else None,
                       isem_a)
        ib = fetch_idx(base_b, ridx_b, cidx_b, sbuf_b if scaled else None,
                       isem_b)
        drain(ia)
        ga = pltpu.async_copy(h_hbm.at[ridx_a], rows_a, gsem_a)
        drain(ib)
        gb = pltpu.async_copy(h_hbm.at[ridx_b], rows_b, gsem_b)
        ga.wait()
        if scaled:
            _scale_rows(rows_a, sbuf_a)
        sa = pltpu.async_copy(rows_a, acc_sh.at[cidx_a], ssem_a, add=True)
        gb.wait()
        if scaled:
            _scale_rows(rows_b, sbuf_b)
        sb = pltpu.async_copy(rows_b, acc_sh.at[cidx_b], ssem_b, add=True)
        sa.wait()
        sb.wait()
    if STEPS % 2:
        base_t = base0 + (STEPS - 1) * K
        drain(fetch_idx(base_t, ridx_a, cidx_a, sbuf_a if scaled else None,
                        isem_a))
        pltpu.async_copy(h_hbm.at[ridx_a], rows_a, gsem_a).wait()
        if scaled:
            _scale_rows(rows_a, sbuf_a)
        pltpu.sync_copy(rows_a, acc_sh.at[cidx_a], add=True)
    plsc.subcore_barrier()
    pltpu.sync_copy(acc_sh.at[pl.ds(sid * slab, slab), :],
                    out_hbm.at[cid, pl.ds(sid * slab, slab), :])


def _attn_body(u_hbm, v_hbm, row_hbm, col_hbm,
               ewc_hbm, ewo_hbm, deg_hbm,
               u_vmem, v_vmem, dc_vmem, do_vmem, ridx, cidx, wc_buf, wo_buf):
    cid = lax.axis_index("c")
    sid = lax.axis_index("s")
    wid = sid * NC + cid
    pltpu.sync_copy(u_hbm, u_vmem)
    pltpu.sync_copy(v_hbm, v_vmem)
    @pl.loop(0, NPAD // 16)
    def _(t):
        dc_vmem[pl.ds(t * 16, 16)] = jnp.zeros((16,), jnp.float32)
        do_vmem[pl.ds(t * 16, 16)] = jnp.zeros((16,), jnp.float32)
    base0 = wid * EW
    @pl.loop(0, STEPS)
    def _(i):
        base = base0 + i * K
        pltpu.sync_copy(row_hbm.at[pl.ds(base, K)], ridx)
        pltpu.sync_copy(col_hbm.at[pl.ds(base, K)], cidx)
        @pl.loop(0, K // 16)
        def _(t):
            r16 = ridx[pl.ds(t * 16, 16)]
            c16 = cidx[pl.ds(t * 16, 16)]
            s = plsc.load_gather(u_vmem, [r16]) + plsc.load_gather(v_vmem, [c16])
            wc = 1.0 / (1.0 + jnp.exp(-s))
            wo = 1.0 - wc
            wc_buf[pl.ds(t * 16, 16)] = wc
            wo_buf[pl.ds(t * 16, 16)] = wo
            plsc.addupdate_scatter(dc_vmem, [r16], wc)
            plsc.addupdate_scatter(do_vmem, [r16], wo)
        pltpu.sync_copy(wc_buf, ewc_hbm.at[pl.ds(base, K)])
        pltpu.sync_copy(wo_buf, ewo_hbm.at[pl.ds(base, K)])
    pltpu.sync_copy(dc_vmem, deg_hbm.at[wid, 0])
    pltpu.sync_copy(do_vmem, deg_hbm.at[wid, 1])


@jax.jit
def _sc_attn(u_pad, v_pad, row, col):
    mesh = plsc.VectorSubcoreMesh(core_axis_name="c", subcore_axis_name="s")
    kern = pl.kernel(
        _attn_body,
        compiler_params=pltpu.CompilerParams(needs_layout_passes=False),
        out_type=(
            jax.ShapeDtypeStruct((E,), jnp.float32),
            jax.ShapeDtypeStruct((E,), jnp.float32),
            jax.ShapeDtypeStruct((NW, 2, NPAD), jnp.float32),
        ),
        mesh=mesh,
        scratch_types=[
            pltpu.VMEM((NPAD,), jnp.float32),
            pltpu.VMEM((NPAD,), jnp.float32),
            pltpu.VMEM((NPAD,), jnp.float32),
            pltpu.VMEM((NPAD,), jnp.float32),
            pltpu.VMEM((K,), jnp.int32),
            pltpu.VMEM((K,), jnp.int32),
            pltpu.VMEM((K,), jnp.float32),
            pltpu.VMEM((K,), jnp.float32),
        ],
    )
    return kern(u_pad, v_pad, row, col)


@jax.jit
def _sc_msgpass(h_pad, row, col):
    mesh = plsc.VectorSubcoreMesh(core_axis_name="c", subcore_axis_name="s")
    kern = pl.kernel(
        functools.partial(_msg_body, False),
        out_type=jax.ShapeDtypeStruct((NC, NPAD, 128), jnp.float32),
        mesh=mesh,
        scratch_types=[
            pltpu.VMEM_SHARED((NPAD, 128), jnp.float32),
            pltpu.VMEM((K,), jnp.int32),
            pltpu.VMEM((K,), jnp.int32),
            pltpu.VMEM((K,), jnp.int32),
            pltpu.VMEM((K,), jnp.int32),
            pltpu.VMEM((K, 128), jnp.float32),
            pltpu.VMEM((K, 128), jnp.float32),
        ] + [pltpu.SemaphoreType.DMA] * 6,
    )
    return kern(h_pad, row, col)


@jax.jit
def _sc_msgpass_scaled(h_pad, row, col, ew):
    mesh = plsc.VectorSubcoreMesh(core_axis_name="c", subcore_axis_name="s")
    kern = pl.kernel(
        functools.partial(_msg_body, True),
        out_type=jax.ShapeDtypeStruct((NC, NPAD, 128), jnp.float32),
        mesh=mesh,
        scratch_types=[
            pltpu.VMEM_SHARED((NPAD, 128), jnp.float32),
            pltpu.VMEM((K,), jnp.int32),
            pltpu.VMEM((K,), jnp.int32),
            pltpu.VMEM((K,), jnp.float32),
            pltpu.VMEM((K,), jnp.int32),
            pltpu.VMEM((K,), jnp.int32),
            pltpu.VMEM((K,), jnp.float32),
            pltpu.VMEM((K, 128), jnp.float32),
            pltpu.VMEM((K, 128), jnp.float32),
        ] + [pltpu.SemaphoreType.DMA] * 6,
    )
    return kern(h_pad, row, col, ew)


def _bn(x, g, b):
    m = jnp.mean(x, axis=0)
    v = jnp.mean(x * x, axis=0) - m * m
    return (x - m) * lax.rsqrt(v + EPS) * g + b


def _log_softmax(z):
    zm = z - jnp.max(z, axis=-1, keepdims=True)
    return zm - jnp.log(jnp.sum(jnp.exp(zm), axis=-1, keepdims=True))


def _head(z, p, pre):
    z = _bn(z, p[pre + "1bn_g"], p[pre + "1bn_b"])
    z = jax.nn.relu(z @ p[pre + "1_W"] + p[pre + "1_b"])
    z = _bn(z, p[pre + "2bn_g"], p[pre + "2bn_b"])
    z = z @ p[pre + "2_W"] + p[pre + "2_b"]
    return _log_softmax(z)


def _pool_heads_body(xc_ref, xo_ref, batch_ref, *rest):
    (hp_refs, outc_ref, outo_ref, outco_ref) = (rest[:-3], rest[-3], rest[-2], rest[-1])
    names = _HEAD_PARAM_NAMES
    p = {k: r[...] for k, r in zip(names, hp_refs)}
    onehot = (batch_ref[0:1, :] == lax.broadcasted_iota(jnp.int32, (G, N), 0))
    onehot = onehot.astype(jnp.float32)
    pc = jnp.dot(onehot, xc_ref[...], preferred_element_type=jnp.float32)
    po = jnp.dot(onehot, xo_ref[...], preferred_element_type=jnp.float32)
    outc_ref[...] = _head(pc, p, "c")
    outo_ref[...] = _head(po, p, "o")
    outco_ref[...] = _head(pc + po, p, "co")


_HEAD_PARAM_NAMES = tuple(
    pre + suf
    for pre in ("c", "o", "co")
    for suf in ("1bn_g", "1bn_b", "1_W", "1_b", "2bn_g", "2bn_b", "2_W", "2_b")
)


def _pool_and_heads(xc, xo, batch, params):
    hp = [params[k] for k in _HEAD_PARAM_NAMES]
    out_shape = [jax.ShapeDtypeStruct((G, C), jnp.float32)] * 3
    outs = pl.pallas_call(
        _pool_heads_body,
        out_shape=out_shape,
    )(xc, xo, batch.reshape(1, N), *hp)
    return outs


def _pad_nodes(h):
    return jnp.pad(h, ((0, NPAD - N), (0, 0)))


def _gcn(x, row_p, col_p, ew, dis, W, b):
    h2 = x @ W
    gs = dis[:, None] * h2
    mp = _sc_msgpass_scaled(_pad_nodes(gs), row_p, col_p, ew)
    out = dis[:, None] * (mp[0, :N] + mp[1, :N] + gs)
    return out + b


def _gin(h, row_p, col_p, p):
    mp = _sc_msgpass(_pad_nodes(h), row_p, col_p)
    h = h + mp[0, :N] + mp[1, :N]
    h = jax.nn.relu(_bn(h @ p["W1"] + p["b1"], p["g1"], p["be1"]))
    return jax.nn.relu(h @ p["W2"] + p["b2"])


def kernel(x, edge_index, batch, params):
    p = params
    row, col = edge_index[0], edge_index[1]
    row_p, col_p = row, col
    h = _bn(x, p["bn_feat_g"], p["bn_feat_b"])
    h = jax.nn.relu(h @ p["conv_feat_W"] + p["conv_feat_b"])
    for lp in p["gin"]:
        h = _gin(h, row_p, col_p, lp)
    wea = p["ea_W"]
    u = h @ (wea[:H, 0] - wea[:H, 1]) + (p["ea_b"][0] - p["ea_b"][1])
    v = h @ (wea[H:, 0] - wea[H:, 1])
    ewc, ewo, degtab = _sc_attn(jnp.pad(u, (0, NPAD - N)),
                                jnp.pad(v, (0, NPAD - N)), row, col)
    degsum = degtab.sum(axis=0)
    dis_c = lax.rsqrt(degsum[0, :N] + 1.0)
    dis_o = lax.rsqrt(degsum[1, :N] + 1.0)
    nl = h @ p["na_W"] + p["na_b"]
    na0 = jax.nn.sigmoid(nl[:, 0] - nl[:, 1])
    xc = na0[:, None] * h
    xo = (1.0 - na0)[:, None] * h
    xc = jax.nn.relu(_gcn(_bn(xc, p["bnc_g"], p["bnc_b"]), row_p, col_p, ewc, dis_c, p["cc_W"], p["cc_b"]))
    xo = jax.nn.relu(_gcn(_bn(xo, p["bno_g"], p["bno_b"]), row_p, col_p, ewo, dis_o, p["oc_W"], p["oc_b"]))
    outc, outo, outco = _pool_and_heads(xc, xo, batch, p)
    return (outc, outo, outco)
